# Initial kernel scaffold; baseline (speedup 1.0000x reference)
import jax
import jax.numpy as jnp
from jax.experimental import pallas as pl


def kernel(node_embs, scorer):
    raise NotImplementedError("write your pallas kernel here")


# speedup vs baseline: 2.1824x; 2.1824x over previous
import functools

import jax
import jax.numpy as jnp
from jax import lax
from jax.experimental import pallas as pl
from jax.experimental.pallas import tpu as pltpu
from jax.experimental.pallas import tpu_sc as plsc

N = 100000
F = 136
K = 5000
NPAD = 100096
CHUNK = NPAD // 16
NV = CHUNK // 16
BINS = 4096
CAND = 102400
SCH = CAND // 16
PBROWS = SCH // 128


def _iota16():
    return lax.iota(jnp.int32, 16)


def _smax(v):
    return lax.reduce_max_p.bind(v, axes=(0,))


def _ukey(k16):
    return plsc.bitcast(k16, jnp.uint32) ^ jnp.uint32(0x80000000)


def _k1_body(x_ref, w_ref, n_ref, o_ref):
    i = pl.program_id(0)
    xv = x_ref[...]
    wv = w_ref[...]
    s = jnp.dot(xv, wv, preferred_element_type=jnp.float32) / n_ref[0, 0]
    b = lax.bitcast_convert_type(s, jnp.int32)
    key = b ^ (lax.shift_right_arithmetic(b, 31) & jnp.int32(0x7FFFFFFF))
    row = lax.broadcasted_iota(jnp.int32, (CHUNK, 1), 0) + i * CHUNK
    o_ref[...] = jnp.where(row < N, key, jnp.int32(-2147483648))


def _k1(x, w, norm):
    return pl.pallas_call(
        _k1_body,
        grid=(NPAD // CHUNK,),
        in_specs=[
            pl.BlockSpec((CHUNK, F), lambda i: (i, 0)),
            pl.BlockSpec((F, 1), lambda i: (0, 0)),
            pl.BlockSpec((1, 1), lambda i: (0, 0), memory_space=pltpu.SMEM),
        ],
        out_specs=pl.BlockSpec((CHUNK, 1), lambda i: (i, 0)),
        out_shape=jax.ShapeDtypeStruct((NPAD, 1), jnp.int32),
    )(x, w, norm)


BUF = CAND + 16


def _k2_body(keys_hbm, oidx_hbm, oval_hbm,
             kb, ub, ib, fb, pb, it2, hl, tb16, cb16, h256, offb, totb, preb,
             t16, g_hist, g_tbl, g_cnt, cu, ci):
    wid = lax.axis_index("s")
    iota = _iota16()
    zeros16 = jnp.zeros((16,), jnp.int32)

    pltpu.sync_copy(keys_hbm.at[pl.ds(wid * CHUNK, CHUNK)], kb)

    def _zero_hl(c, _):
        hl[pl.ds(c * 16, 16)] = zeros16
        return _
    lax.fori_loop(0, BINS // 16, _zero_hl, None)

    def _fill_it2(c, _):
        for j in range(8):
            it2[c, pl.ds(16 * j, 16)] = iota + (16 * j) + c * 128
        return _
    lax.fori_loop(0, 32, _fill_it2, None)

    @pl.when(wid == 0)
    def _():
        pltpu.sync_copy(hl, g_hist)
    plsc.subcore_barrier()

    ones16 = zeros16 + 1

    def _hist(v, _):
        k16 = kb[pl.ds(v * 16, 16)]
        uk = _ukey(k16)
        bin_ = (uk >> jnp.uint32(20)).astype(jnp.int32)
        plsc.addupdate_scatter(hl, [bin_], ones16)
        return _
    lax.fori_loop(0, NV, _hist, None)

    def _merge(c, _):
        pltpu.sync_copy(hl.at[pl.ds(c * 128, 128)],
                        g_hist.at[it2.at[c]], add=True)
        return _
    lax.fori_loop(0, 32, _merge, None)
    plsc.subcore_barrier()

    pltpu.sync_copy(g_hist, hl)

    def _scan(i, carry):
        cnt_so_far, bt = carry
        c = (BINS // 16 - 1) - i
        ch = hl[pl.ds(c * 16, 16)]
        rv = lax.rev(ch, (0,))
        rc = plsc.cumsum(rv)
        tot = _smax(rc)
        sel = (cnt_so_far + rc) >= K
        cross = jnp.logical_and(cnt_so_far < K, cnt_so_far + tot >= K)
        p = _smax(plsc.all_reduce_ffs(sel))
        bt_c = 16 * c + 15 - p
        bt = jnp.where(cross, bt_c, bt)
        return cnt_so_far + tot, bt
    _, bT = lax.fori_loop(0, BINS // 16, _scan, (jnp.int32(0), jnp.int32(0)))
    ut_lo = bT.astype(jnp.uint32) << jnp.uint32(20)

    def _compact(v, cur):
        k16 = kb[pl.ds(v * 16, 16)]
        uk = _ukey(k16)
        m = uk >= ut_lo
        idxv = iota + (wid * CHUNK + v * 16)
        plsc.store_compressed(ub.at[pl.ds(cur, 16)], uk, mask=m)
        plsc.store_compressed(ib.at[pl.ds(cur, 16)], idxv, mask=m)
        return cur + _smax(plsc.all_reduce_population_count(m))
    n_t = lax.fori_loop(0, NV, _compact, jnp.int32(0))

    t16[...] = zeros16 + n_t
    pltpu.sync_copy(t16, g_cnt.at[wid])
    plsc.subcore_barrier()
    pltpu.sync_copy(g_cnt, cb16)
    my_off = jnp.int32(0)
    S = jnp.int32(0)
    for j in range(16):
        nj = _smax(cb16[j, :])
        my_off = my_off + jnp.where(jnp.int32(j) < wid, nj, 0)
        S = S + nj

    def _scat(c, _):
        for j in range(8):
            li = c * 128 + 16 * j + iota
            valid = li < n_t
            pos = jnp.where(valid, my_off + li, CAND + iota)
            pb[c, pl.ds(16 * j, 16)] = pos
        pltpu.sync_copy(ub.at[pl.ds(c * 128, 128)], cu.at[pb.at[c]])
        pltpu.sync_copy(ib.at[pl.ds(c * 128, 128)], ci.at[pb.at[c]])
        return _
    lax.fori_loop(0, (n_t + 127) // 128, _scat, None)
    plsc.subcore_barrier()

    cs = ((S + 15) // 16 + 127) // 128 * 128
    a_t = wid * cs
    v_t = jnp.clip(S - a_t, 0, cs)
    nl = (v_t + 127) // 128
    nv2 = (v_t + 15) // 16

    def _pass(p, _):
        par = p % 2
        src0 = par * BUF
        dst0 = (1 - par) * BUF
        shift = (p * 8).astype(jnp.uint32)

        def _load(jc, _):
            pltpu.sync_copy(cu.at[pl.ds(src0 + a_t + jc * 128, 128)],
                            ub.at[pl.ds(jc * 128, 128)])
            pltpu.sync_copy(ci.at[pl.ds(src0 + a_t + jc * 128, 128)],
                            ib.at[pl.ds(jc * 128, 128)])
            return _
        lax.fori_loop(0, nl, _load, None)

        def _zero_h(c, _):
            h256[pl.ds(c * 16, 16)] = zeros16
            return _
        lax.fori_loop(0, 16, _zero_h, None)

        def _hist2(v, _):
            uk = ub[pl.ds(v * 16, 16)]
            d = ((uk >> shift) & jnp.uint32(0xFF)).astype(jnp.int32)
            valid = (v * 16 + iota) < v_t
            plsc.addupdate_scatter(h256, [d], jnp.where(valid, 1, 0))
            return _
        lax.fori_loop(0, nv2, _hist2, None)

        pltpu.sync_copy(h256.at[pl.ds(0, 256)], g_tbl.at[wid])
        plsc.subcore_barrier()
        pltpu.sync_copy(g_tbl, tb16)

        def _totpre(c, _):
            acc_t = zeros16
            acc_p = zeros16
            for j in range(16):
                row = tb16[j, pl.ds(c * 16, 16)]
                acc_t = acc_t + row
                acc_p = acc_p + jnp.where(jnp.int32(j) < wid, row, zeros16)
            totb[pl.ds(c * 16, 16)] = acc_t
            preb[pl.ds(c * 16, 16)] = acc_p
            return _
        lax.fori_loop(0, 16, _totpre, None)

        def _suff(i, carry):
            c = 15 - i
            ch = totb[pl.ds(c * 16, 16)]
            rv = lax.rev(ch, (0,))
            rc = plsc.cumsum(rv)
            excl_rev = carry + rc - rv
            excl = lax.rev(excl_rev, (0,))
            offb[pl.ds(c * 16, 16)] = excl + preb[pl.ds(c * 16, 16)]
            return carry + _smax(rc)
        lax.fori_loop(0, 16, _suff, jnp.int32(0))

        def _rank(v, _):
            uk = ub[pl.ds(v * 16, 16)]
            d = ((uk >> shift) & jnp.uint32(0xFF)).astype(jnp.int32)
            valid = (v * 16 + iota) < v_t
            d_aug = jnp.where(valid, d, 256 + iota)
            cnt, lastm = plsc.scan_count(d_aug)
            start = plsc.load_gather(offb, [d_aug])
            pos = jnp.where(valid, dst0 + start + cnt - 1,
                            dst0 + CAND + iota)
            pb[v // 8, pl.ds((v % 8) * 16, 16)] = pos
            plsc.addupdate_scatter(offb, [d_aug],
                                   jnp.where(jnp.logical_and(lastm, valid),
                                             cnt, 0))
            return _
        lax.fori_loop(0, nl * 8, _rank, None)

        def _scat2(c, _):
            pltpu.sync_copy(ub.at[pl.ds(c * 128, 128)], cu.at[pb.at[c]])
            pltpu.sync_copy(ib.at[pl.ds(c * 128, 128)], ci.at[pb.at[c]])
            return _
        lax.fori_loop(0, nl, _scat2, None)
        plsc.subcore_barrier()
        return _
    lax.fori_loop(0, 4, _pass, None)

    base = wid * 320
    pltpu.sync_copy(cu.at[pl.ds(base, 320)], ub.at[pl.ds(0, 320)])
    pltpu.sync_copy(ci.at[pl.ds(base, 320)], ib.at[pl.ds(0, 320)])

    def _conv(v, _):
        uk = ub[pl.ds(v * 16, 16)]
        kk = plsc.bitcast(uk ^ jnp.uint32(0x80000000), jnp.int32)
        bits = jnp.where(kk >= 0, kk, kk ^ jnp.int32(0x7FFFFFFF))
        fb[pl.ds(v * 16, 16)] = plsc.bitcast(bits, jnp.float32)
        gpos = base + v * 16 + iota
        iv = ib[pl.ds(v * 16, 16)]
        ib[pl.ds(v * 16, 16)] = jnp.where(gpos < K, iv, 0)
        return _
    lax.fori_loop(0, 20, _conv, None)

    @pl.when(wid < 15)
    def _():
        pltpu.sync_copy(fb, oval_hbm.at[pl.ds(base, 320)])
        pltpu.sync_copy(ib.at[pl.ds(0, 320)], oidx_hbm.at[pl.ds(base, 320)])

    @pl.when(wid == 15)
    def _():
        pltpu.sync_copy(fb.at[pl.ds(0, 200)], oval_hbm.at[pl.ds(4800, 200)])
        pltpu.sync_copy(ib.at[pl.ds(0, 200)], oidx_hbm.at[pl.ds(4800, 200)])


def _k2(keys):
    mesh = plsc.VectorSubcoreMesh(
        core_axis_name="c", subcore_axis_name="s", num_cores=1)
    f = functools.partial(
        pl.kernel,
        mesh=mesh,
        compiler_params=pltpu.CompilerParams(needs_layout_passes=False),
        out_type=[jax.ShapeDtypeStruct((K,), jnp.int32),
                  jax.ShapeDtypeStruct((K,), jnp.float32)],
        scratch_types=[
            pltpu.VMEM((CHUNK,), jnp.int32),
            pltpu.VMEM((SCH,), jnp.uint32),
            pltpu.VMEM((SCH,), jnp.int32),
            pltpu.VMEM((320,), jnp.float32),
            pltpu.VMEM((PBROWS, 128), jnp.int32),
            pltpu.VMEM((32, 128), jnp.int32),
            pltpu.VMEM((BINS,), jnp.int32),
            pltpu.VMEM((16, 256), jnp.int32),
            pltpu.VMEM((16, 16), jnp.int32),
            pltpu.VMEM((272,), jnp.int32),
            pltpu.VMEM((272,), jnp.int32),
            pltpu.VMEM((256,), jnp.int32),
            pltpu.VMEM((256,), jnp.int32),
            pltpu.VMEM((16,), jnp.int32),
            pltpu.VMEM_SHARED((BINS,), jnp.int32),
            pltpu.VMEM_SHARED((16, 256), jnp.int32),
            pltpu.VMEM_SHARED((16, 16), jnp.int32),
            pltpu.VMEM_SHARED((2 * BUF,), jnp.uint32),
            pltpu.VMEM_SHARED((2 * BUF,), jnp.int32),
        ],
    )(_k2_body)
    return f(keys)


def _k3_body(emb_hbm, idx_hbm, out_hbm, ix40, rb, sem):
    wid = lax.axis_index("c") * 16 + lax.axis_index("s")
    iota = _iota16()
    for c in range(4):
        ci = wid * 4 + c

        @pl.when(ci < K // 40)
        def _():
            pltpu.sync_copy(idx_hbm.at[pl.ds(ci * 40, 40)],
                            ix40.at[pl.ds(0, 40)])
            copies = []
            for r in range(40):
                v = ix40[pl.ds((r // 16) * 16, 16)]
                row = _smax(jnp.where(iota == (r % 16), v, 0))
                copies.append(
                    pltpu.async_copy(emb_hbm.at[row], rb.at[r], sem))
            for d in copies:
                d.wait()
            pltpu.sync_copy(rb, out_hbm.at[pl.ds(ci * 40, 40)])


def _k3(emb, idx):
    mesh = plsc.VectorSubcoreMesh(
        core_axis_name="c", subcore_axis_name="s", num_cores=2)
    f = functools.partial(
        pl.kernel,
        mesh=mesh,
        compiler_params=pltpu.CompilerParams(needs_layout_passes=False),
        out_type=jax.ShapeDtypeStruct((K, F), jnp.float32),
        scratch_types=[
            pltpu.VMEM((48,), jnp.int32),
            pltpu.VMEM((40, F), jnp.float32),
            pltpu.SemaphoreType.DMA,
        ],
    )(_k3_body)
    return f(emb, idx)


def _k4_body(r_ref, v_ref, o_ref):
    g = jnp.tanh(v_ref[...])
    o_ref[...] = jnp.transpose(r_ref[...] * g, (1, 0))


def _k4(rows, vals):
    return pl.pallas_call(
        _k4_body,
        in_specs=[
            pl.BlockSpec((K, F), lambda: (0, 0)),
            pl.BlockSpec((K, 1), lambda: (0, 0)),
        ],
        out_specs=pl.BlockSpec((F, K), lambda: (0, 0)),
        out_shape=jax.ShapeDtypeStruct((F, K), jnp.float32),
    )(rows, vals)


def kernel(node_embs, scorer):
    norm = jnp.linalg.norm(scorer).reshape(1, 1)
    keys = _k1(node_embs, scorer, norm).reshape(NPAD)
    oidx, oval = _k2(keys)
    rows = _k3(node_embs, oidx)
    return _k4(rows, oval.reshape(K, 1))


# speedup vs baseline: 2.1825x; 1.0000x over previous
import functools

import jax
import jax.numpy as jnp
from jax import lax
from jax.experimental import pallas as pl
from jax.experimental.pallas import tpu as pltpu
from jax.experimental.pallas import tpu_sc as plsc

N = 100000
F = 136
K = 5000
NPAD = 100096
CHUNK = NPAD // 16
NV = CHUNK // 16
BINS = 4096
CAND = 102400
SCH = CAND // 16
PBROWS = SCH // 128


def _iota16():
    return lax.iota(jnp.int32, 16)


def _smax(v):
    return lax.reduce_max_p.bind(v, axes=(0,))


def _ukey(k16):
    return plsc.bitcast(k16, jnp.uint32) ^ jnp.uint32(0x80000000)


def _k1_body(x_ref, w_ref, n_ref, o_ref):
    i = pl.program_id(0)
    xv = x_ref[...]
    wv = w_ref[...]
    s = jnp.dot(xv, wv, preferred_element_type=jnp.float32) / n_ref[0, 0]
    b = lax.bitcast_convert_type(s, jnp.int32)
    key = b ^ (lax.shift_right_arithmetic(b, 31) & jnp.int32(0x7FFFFFFF))
    row = lax.broadcasted_iota(jnp.int32, (CHUNK, 1), 0) + i * CHUNK
    o_ref[...] = jnp.where(row < N, key, jnp.int32(-2147483648))


def _k1(x, w, norm):
    return pl.pallas_call(
        _k1_body,
        grid=(NPAD // CHUNK,),
        in_specs=[
            pl.BlockSpec((CHUNK, F), lambda i: (i, 0)),
            pl.BlockSpec((F, 1), lambda i: (0, 0)),
            pl.BlockSpec((1, 1), lambda i: (0, 0), memory_space=pltpu.SMEM),
        ],
        out_specs=pl.BlockSpec((CHUNK, 1), lambda i: (i, 0)),
        out_shape=jax.ShapeDtypeStruct((NPAD, 1), jnp.int32),
    )(x, w, norm)


BUF = CAND + 16


def _k2_body(keys_hbm, oidx_hbm, oval_hbm,
             kb, ub, ib, fb, pb, it2, hl, tb16, cb16, h256, offb, totb, preb,
             t16, g_hist, g_tbl, g_cnt, cu, ci):
    wid = lax.axis_index("s")
    iota = _iota16()
    zeros16 = jnp.zeros((16,), jnp.int32)

    pltpu.sync_copy(keys_hbm.at[pl.ds(wid * CHUNK, CHUNK)], kb)

    def _zero_hl(c, _):
        hl[pl.ds(c * 16, 16)] = zeros16
        return _
    lax.fori_loop(0, BINS // 16, _zero_hl, None)

    def _fill_it2(c, _):
        for j in range(8):
            it2[c, pl.ds(16 * j, 16)] = iota + (16 * j) + c * 128
        return _
    lax.fori_loop(0, 32, _fill_it2, None)

    @pl.when(wid == 0)
    def _():
        pltpu.sync_copy(hl, g_hist)
    plsc.subcore_barrier()

    ones16 = zeros16 + 1

    def _hist(v, _):
        k16 = kb[pl.ds(v * 16, 16)]
        uk = _ukey(k16)
        bin_ = (uk >> jnp.uint32(20)).astype(jnp.int32)
        plsc.addupdate_scatter(hl, [bin_], ones16)
        return _
    lax.fori_loop(0, NV, _hist, None)

    def _merge(c, _):
        pltpu.sync_copy(hl.at[pl.ds(c * 128, 128)],
                        g_hist.at[it2.at[c]], add=True)
        return _
    lax.fori_loop(0, 32, _merge, None)
    plsc.subcore_barrier()

    pltpu.sync_copy(g_hist, hl)

    def _scan(i, carry):
        cnt_so_far, bt = carry
        c = (BINS // 16 - 1) - i
        ch = hl[pl.ds(c * 16, 16)]
        rv = lax.rev(ch, (0,))
        rc = plsc.cumsum(rv)
        tot = _smax(rc)
        sel = (cnt_so_far + rc) >= K
        cross = jnp.logical_and(cnt_so_far < K, cnt_so_far + tot >= K)
        p = _smax(plsc.all_reduce_ffs(sel))
        bt_c = 16 * c + 15 - p
        bt = jnp.where(cross, bt_c, bt)
        return cnt_so_far + tot, bt
    _, bT = lax.fori_loop(0, BINS // 16, _scan, (jnp.int32(0), jnp.int32(0)))
    ut_lo = bT.astype(jnp.uint32) << jnp.uint32(20)

    def _compact(v, cur):
        k16 = kb[pl.ds(v * 16, 16)]
        uk = _ukey(k16)
        m = uk >= ut_lo
        idxv = iota + (wid * CHUNK + v * 16)
        plsc.store_compressed(ub.at[pl.ds(cur, 16)], uk, mask=m)
        plsc.store_compressed(ib.at[pl.ds(cur, 16)], idxv, mask=m)
        return cur + _smax(plsc.all_reduce_population_count(m))
    n_t = lax.fori_loop(0, NV, _compact, jnp.int32(0))

    t16[...] = zeros16 + n_t
    pltpu.sync_copy(t16, g_cnt.at[wid])
    plsc.subcore_barrier()
    pltpu.sync_copy(g_cnt, cb16)
    my_off = jnp.int32(0)
    S = jnp.int32(0)
    for j in range(16):
        nj = _smax(cb16[j, :])
        my_off = my_off + jnp.where(jnp.int32(j) < wid, nj, 0)
        S = S + nj

    def _scat(c, _):
        for j in range(8):
            li = c * 128 + 16 * j + iota
            valid = li < n_t
            pos = jnp.where(valid, my_off + li, CAND + iota)
            pb[c, pl.ds(16 * j, 16)] = pos
        pltpu.sync_copy(ub.at[pl.ds(c * 128, 128)], cu.at[pb.at[c]])
        pltpu.sync_copy(ib.at[pl.ds(c * 128, 128)], ci.at[pb.at[c]])
        return _
    lax.fori_loop(0, (n_t + 127) // 128, _scat, None)
    plsc.subcore_barrier()

    cs = ((S + 15) // 16 + 127) // 128 * 128
    a_t = wid * cs
    v_t = jnp.clip(S - a_t, 0, cs)
    nl = (v_t + 127) // 128
    nv2 = (v_t + 15) // 16

    def _pass(p, _):
        par = p % 2
        src0 = par * BUF
        dst0 = (1 - par) * BUF
        shift = (p * 8).astype(jnp.uint32)

        def _load(jc, _):
            pltpu.sync_copy(cu.at[pl.ds(src0 + a_t + jc * 128, 128)],
                            ub.at[pl.ds(jc * 128, 128)])
            pltpu.sync_copy(ci.at[pl.ds(src0 + a_t + jc * 128, 128)],
                            ib.at[pl.ds(jc * 128, 128)])
            return _
        lax.fori_loop(0, nl, _load, None)

        def _zero_h(c, _):
            h256[pl.ds(c * 16, 16)] = zeros16
            return _
        lax.fori_loop(0, 16, _zero_h, None)

        def _hist2(v, _):
            uk = ub[pl.ds(v * 16, 16)]
            d = ((uk >> shift) & jnp.uint32(0xFF)).astype(jnp.int32)
            valid = (v * 16 + iota) < v_t
            plsc.addupdate_scatter(h256, [d], jnp.where(valid, 1, 0))
            return _
        lax.fori_loop(0, nv2, _hist2, None)

        pltpu.sync_copy(h256.at[pl.ds(0, 256)], g_tbl.at[wid])
        plsc.subcore_barrier()
        pltpu.sync_copy(g_tbl, tb16)

        def _totpre(c, _):
            acc_t = zeros16
            acc_p = zeros16
            for j in range(16):
                row = tb16[j, pl.ds(c * 16, 16)]
                acc_t = acc_t + row
                acc_p = acc_p + jnp.where(jnp.int32(j) < wid, row, zeros16)
            totb[pl.ds(c * 16, 16)] = acc_t
            preb[pl.ds(c * 16, 16)] = acc_p
            return _
        lax.fori_loop(0, 16, _totpre, None)

        def _suff(i, carry):
            c = 15 - i
            ch = totb[pl.ds(c * 16, 16)]
            rv = lax.rev(ch, (0,))
            rc = plsc.cumsum(rv)
            excl_rev = carry + rc - rv
            excl = lax.rev(excl_rev, (0,))
            offb[pl.ds(c * 16, 16)] = excl + preb[pl.ds(c * 16, 16)]
            return carry + _smax(rc)
        lax.fori_loop(0, 16, _suff, jnp.int32(0))

        def _rank(v, _):
            uk = ub[pl.ds(v * 16, 16)]
            d = ((uk >> shift) & jnp.uint32(0xFF)).astype(jnp.int32)
            valid = (v * 16 + iota) < v_t
            d_aug = jnp.where(valid, d, 256 + iota)
            cnt, lastm = plsc.scan_count(d_aug)
            start = plsc.load_gather(offb, [d_aug])
            pos = jnp.where(valid, dst0 + start + cnt - 1,
                            dst0 + CAND + iota)
            pb[v // 8, pl.ds((v % 8) * 16, 16)] = pos
            plsc.addupdate_scatter(offb, [d_aug],
                                   jnp.where(jnp.logical_and(lastm, valid),
                                             cnt, 0))
            return _
        lax.fori_loop(0, nl * 8, _rank, None)

        def _scat2(c, _):
            pltpu.sync_copy(ub.at[pl.ds(c * 128, 128)], cu.at[pb.at[c]])
            pltpu.sync_copy(ib.at[pl.ds(c * 128, 128)], ci.at[pb.at[c]])
            return _
        lax.fori_loop(0, nl, _scat2, None)
        plsc.subcore_barrier()
        return _
    lax.fori_loop(0, 4, _pass, None)

    base = wid * 320
    pltpu.sync_copy(cu.at[pl.ds(base, 320)], ub.at[pl.ds(0, 320)])
    pltpu.sync_copy(ci.at[pl.ds(base, 320)], ib.at[pl.ds(0, 320)])

    def _conv(v, _):
        uk = ub[pl.ds(v * 16, 16)]
        kk = plsc.bitcast(uk ^ jnp.uint32(0x80000000), jnp.int32)
        bits = jnp.where(kk >= 0, kk, kk ^ jnp.int32(0x7FFFFFFF))
        fb[pl.ds(v * 16, 16)] = plsc.bitcast(bits, jnp.float32)
        gpos = base + v * 16 + iota
        iv = ib[pl.ds(v * 16, 16)]
        ib[pl.ds(v * 16, 16)] = jnp.where(gpos < K, iv, 0)
        return _
    lax.fori_loop(0, 20, _conv, None)

    @pl.when(wid < 15)
    def _():
        pltpu.sync_copy(fb, oval_hbm.at[pl.ds(base, 320)])
        pltpu.sync_copy(ib.at[pl.ds(0, 320)], oidx_hbm.at[pl.ds(base, 320)])

    @pl.when(wid == 15)
    def _():
        pltpu.sync_copy(fb.at[pl.ds(0, 200)], oval_hbm.at[pl.ds(4800, 200)])
        pltpu.sync_copy(ib.at[pl.ds(0, 200)], oidx_hbm.at[pl.ds(4800, 200)])


def _k2(keys):
    mesh = plsc.VectorSubcoreMesh(
        core_axis_name="c", subcore_axis_name="s", num_cores=1)
    f = functools.partial(
        pl.kernel,
        mesh=mesh,
        compiler_params=pltpu.CompilerParams(needs_layout_passes=False),
        out_type=[jax.ShapeDtypeStruct((K,), jnp.int32),
                  jax.ShapeDtypeStruct((K,), jnp.float32)],
        scratch_types=[
            pltpu.VMEM((CHUNK,), jnp.int32),
            pltpu.VMEM((SCH,), jnp.uint32),
            pltpu.VMEM((SCH,), jnp.int32),
            pltpu.VMEM((320,), jnp.float32),
            pltpu.VMEM((PBROWS, 128), jnp.int32),
            pltpu.VMEM((32, 128), jnp.int32),
            pltpu.VMEM((BINS,), jnp.int32),
            pltpu.VMEM((16, 256), jnp.int32),
            pltpu.VMEM((16, 16), jnp.int32),
            pltpu.VMEM((272,), jnp.int32),
            pltpu.VMEM((272,), jnp.int32),
            pltpu.VMEM((256,), jnp.int32),
            pltpu.VMEM((256,), jnp.int32),
            pltpu.VMEM((16,), jnp.int32),
            pltpu.VMEM_SHARED((BINS,), jnp.int32),
            pltpu.VMEM_SHARED((16, 256), jnp.int32),
            pltpu.VMEM_SHARED((16, 16), jnp.int32),
            pltpu.VMEM_SHARED((2 * BUF,), jnp.uint32),
            pltpu.VMEM_SHARED((2 * BUF,), jnp.int32),
        ],
    )(_k2_body)
    return f(keys)


def _k3_body(emb_hbm, idx_hbm, out_hbm, ix40, rb, sem):
    wid = lax.axis_index("c") * 16 + lax.axis_index("s")
    iota = _iota16()
    for c in range(4):
        ci = wid * 4 + c

        @pl.when(ci < K // 40)
        def _():
            pltpu.sync_copy(idx_hbm.at[pl.ds(ci * 40, 40)],
                            ix40.at[pl.ds(0, 40)])
            copies = []
            for r in range(40):
                v = ix40[pl.ds((r // 16) * 16, 16)]
                row = _smax(jnp.where(iota == (r % 16), v, 0))
                copies.append(
                    pltpu.async_copy(emb_hbm.at[row], rb.at[r], sem))
            for d in copies:
                d.wait()
            pltpu.sync_copy(rb, out_hbm.at[pl.ds(ci * 40, 40)])


def _k3(emb, idx):
    mesh = plsc.VectorSubcoreMesh(
        core_axis_name="c", subcore_axis_name="s", num_cores=2)
    f = functools.partial(
        pl.kernel,
        mesh=mesh,
        compiler_params=pltpu.CompilerParams(needs_layout_passes=False,
                                             use_tc_tiling_on_sc=True),
        out_type=jax.ShapeDtypeStruct((K, F), jnp.float32),
        scratch_types=[
            pltpu.VMEM((48,), jnp.int32),
            pltpu.VMEM((40, F), jnp.float32),
            pltpu.SemaphoreType.DMA,
        ],
    )(_k3_body)
    return f(emb, idx)


def _k4_body(r_ref, v_ref, o_ref):
    g = jnp.tanh(v_ref[...])
    o_ref[...] = jnp.transpose(r_ref[...] * g, (1, 0))


def _k4(rows, vals):
    return pl.pallas_call(
        _k4_body,
        in_specs=[
            pl.BlockSpec((K, F), lambda: (0, 0)),
            pl.BlockSpec((K, 1), lambda: (0, 0)),
        ],
        out_specs=pl.BlockSpec((F, K), lambda: (0, 0)),
        out_shape=jax.ShapeDtypeStruct((F, K), jnp.float32),
    )(rows, vals)


def kernel(node_embs, scorer):
    norm = jnp.linalg.norm(scorer).reshape(1, 1)
    keys = _k1(node_embs, scorer, norm).reshape(NPAD)
    oidx, oval = _k2(keys)
    rows = _k3(node_embs, oidx)
    return _k4(rows, oval.reshape(K, 1))


# speedup vs baseline: 2.9274x; 1.3413x over previous
import functools

import jax
import jax.numpy as jnp
from jax import lax
from jax.experimental import pallas as pl
from jax.experimental.pallas import tpu as pltpu
from jax.experimental.pallas import tpu_sc as plsc

N = 100000
F = 136
K = 5000
NPAD = 100096
CHUNK = NPAD // 16
NV = CHUNK // 16
BINS = 4096
CAND = 102400
SCH = CAND // 16
PBROWS = SCH // 128


def _iota16():
    return lax.iota(jnp.int32, 16)


def _smax(v):
    return lax.reduce_max_p.bind(v, axes=(0,))


def _ukey(k16):
    return plsc.bitcast(k16, jnp.uint32) ^ jnp.uint32(0x80000000)


K1B = 5888


def _k1_body(xt_ref, w_ref, n_ref, o_ref, xc_ref):
    i = pl.program_id(0)
    xv = xt_ref[...]
    wv = w_ref[...]
    s = lax.dot_general(xv, wv, (((0,), (0,)), ((), ())),
                        preferred_element_type=jnp.float32) / n_ref[0, 0]
    b = lax.bitcast_convert_type(s, jnp.int32)
    key = b ^ (lax.shift_right_arithmetic(b, 31) & jnp.int32(0x7FFFFFFF))
    row = lax.broadcasted_iota(jnp.int32, (K1B, 1), 0) + i * K1B
    o_ref[...] = jnp.where(row < N, key, jnp.int32(-2147483648))
    xc_ref[...] = jnp.transpose(xv, (1, 0))


def _k1(xt, w, norm):
    return pl.pallas_call(
        _k1_body,
        grid=(NPAD // K1B,),
        in_specs=[
            pl.BlockSpec((F, K1B), lambda i: (0, i)),
            pl.BlockSpec((F, 1), lambda i: (0, 0)),
            pl.BlockSpec((1, 1), lambda i: (0, 0), memory_space=pltpu.SMEM),
        ],
        out_specs=[pl.BlockSpec((K1B, 1), lambda i: (i, 0)),
                   pl.BlockSpec((K1B, F), lambda i: (i, 0))],
        out_shape=[jax.ShapeDtypeStruct((NPAD, 1), jnp.int32),
                   jax.ShapeDtypeStruct((NPAD, F), jnp.float32)],
    )(xt, w, norm)


BUF = CAND + 16


def _k2_body(keys_hbm, oidx_hbm, oval_hbm,
             kb, ub, ib, fb, pb, it2, hl, tb16, cb16, h256, offb, totb, preb,
             t16, g_hist, g_tbl, g_cnt, cu, ci):
    wid = lax.axis_index("s")
    iota = _iota16()
    zeros16 = jnp.zeros((16,), jnp.int32)

    pltpu.sync_copy(keys_hbm.at[pl.ds(wid * CHUNK, CHUNK)], kb)

    def _zero_hl(c, _):
        hl[pl.ds(c * 16, 16)] = zeros16
        return _
    lax.fori_loop(0, BINS // 16, _zero_hl, None)

    def _fill_it2(c, _):
        for j in range(8):
            it2[c, pl.ds(16 * j, 16)] = iota + (16 * j) + c * 128
        return _
    lax.fori_loop(0, 32, _fill_it2, None)

    @pl.when(wid == 0)
    def _():
        pltpu.sync_copy(hl, g_hist)
    plsc.subcore_barrier()

    ones16 = zeros16 + 1

    def _hist(v, _):
        k16 = kb[pl.ds(v * 16, 16)]
        uk = _ukey(k16)
        bin_ = (uk >> jnp.uint32(20)).astype(jnp.int32)
        plsc.addupdate_scatter(hl, [bin_], ones16)
        return _
    lax.fori_loop(0, NV, _hist, None)

    def _merge(c, _):
        pltpu.sync_copy(hl.at[pl.ds(c * 128, 128)],
                        g_hist.at[it2.at[c]], add=True)
        return _
    lax.fori_loop(0, 32, _merge, None)
    plsc.subcore_barrier()

    pltpu.sync_copy(g_hist, hl)

    def _scan(i, carry):
        cnt_so_far, bt = carry
        c = (BINS // 16 - 1) - i
        ch = hl[pl.ds(c * 16, 16)]
        rv = lax.rev(ch, (0,))
        rc = plsc.cumsum(rv)
        tot = _smax(rc)
        sel = (cnt_so_far + rc) >= K
        cross = jnp.logical_and(cnt_so_far < K, cnt_so_far + tot >= K)
        p = _smax(plsc.all_reduce_ffs(sel))
        bt_c = 16 * c + 15 - p
        bt = jnp.where(cross, bt_c, bt)
        return cnt_so_far + tot, bt
    _, bT = lax.fori_loop(0, BINS // 16, _scan, (jnp.int32(0), jnp.int32(0)))
    ut_lo = bT.astype(jnp.uint32) << jnp.uint32(20)

    def _compact(v, cur):
        k16 = kb[pl.ds(v * 16, 16)]
        uk = _ukey(k16)
        m = uk >= ut_lo
        idxv = iota + (wid * CHUNK + v * 16)
        plsc.store_compressed(ub.at[pl.ds(cur, 16)], uk, mask=m)
        plsc.store_compressed(ib.at[pl.ds(cur, 16)], idxv, mask=m)
        return cur + _smax(plsc.all_reduce_population_count(m))
    n_t = lax.fori_loop(0, NV, _compact, jnp.int32(0))

    t16[...] = zeros16 + n_t
    pltpu.sync_copy(t16, g_cnt.at[wid])
    plsc.subcore_barrier()
    pltpu.sync_copy(g_cnt, cb16)
    my_off = jnp.int32(0)
    S = jnp.int32(0)
    for j in range(16):
        nj = _smax(cb16[j, :])
        my_off = my_off + jnp.where(jnp.int32(j) < wid, nj, 0)
        S = S + nj

    def _scat(c, _):
        for j in range(8):
            li = c * 128 + 16 * j + iota
            valid = li < n_t
            pos = jnp.where(valid, my_off + li, CAND + iota)
            pb[c, pl.ds(16 * j, 16)] = pos
        pltpu.sync_copy(ub.at[pl.ds(c * 128, 128)], cu.at[pb.at[c]])
        pltpu.sync_copy(ib.at[pl.ds(c * 128, 128)], ci.at[pb.at[c]])
        return _
    lax.fori_loop(0, (n_t + 127) // 128, _scat, None)
    plsc.subcore_barrier()

    cs = ((S + 15) // 16 + 127) // 128 * 128
    a_t = wid * cs
    v_t = jnp.clip(S - a_t, 0, cs)
    nl = (v_t + 127) // 128
    nv2 = (v_t + 15) // 16

    def _pass(p, _):
        par = p % 2
        src0 = par * BUF
        dst0 = (1 - par) * BUF
        shift = (p * 8).astype(jnp.uint32)

        def _load(jc, _):
            pltpu.sync_copy(cu.at[pl.ds(src0 + a_t + jc * 128, 128)],
                            ub.at[pl.ds(jc * 128, 128)])
            pltpu.sync_copy(ci.at[pl.ds(src0 + a_t + jc * 128, 128)],
                            ib.at[pl.ds(jc * 128, 128)])
            return _
        lax.fori_loop(0, nl, _load, None)

        def _zero_h(c, _):
            h256[pl.ds(c * 16, 16)] = zeros16
            return _
        lax.fori_loop(0, 16, _zero_h, None)

        def _hist2(v, _):
            uk = ub[pl.ds(v * 16, 16)]
            d = ((uk >> shift) & jnp.uint32(0xFF)).astype(jnp.int32)
            valid = (v * 16 + iota) < v_t
            plsc.addupdate_scatter(h256, [d], jnp.where(valid, 1, 0))
            return _
        lax.fori_loop(0, nv2, _hist2, None)

        pltpu.sync_copy(h256.at[pl.ds(0, 256)], g_tbl.at[wid])
        plsc.subcore_barrier()
        pltpu.sync_copy(g_tbl, tb16)

        def _totpre(c, _):
            acc_t = zeros16
            acc_p = zeros16
            for j in range(16):
                row = tb16[j, pl.ds(c * 16, 16)]
                acc_t = acc_t + row
                acc_p = acc_p + jnp.where(jnp.int32(j) < wid, row, zeros16)
            totb[pl.ds(c * 16, 16)] = acc_t
            preb[pl.ds(c * 16, 16)] = acc_p
            return _
        lax.fori_loop(0, 16, _totpre, None)

        def _suff(i, carry):
            c = 15 - i
            ch = totb[pl.ds(c * 16, 16)]
            rv = lax.rev(ch, (0,))
            rc = plsc.cumsum(rv)
            excl_rev = carry + rc - rv
            excl = lax.rev(excl_rev, (0,))
            offb[pl.ds(c * 16, 16)] = excl + preb[pl.ds(c * 16, 16)]
            return carry + _smax(rc)
        lax.fori_loop(0, 16, _suff, jnp.int32(0))

        def _rank(v, _):
            uk = ub[pl.ds(v * 16, 16)]
            d = ((uk >> shift) & jnp.uint32(0xFF)).astype(jnp.int32)
            valid = (v * 16 + iota) < v_t
            d_aug = jnp.where(valid, d, 256 + iota)
            cnt, lastm = plsc.scan_count(d_aug)
            start = plsc.load_gather(offb, [d_aug])
            pos = jnp.where(valid, dst0 + start + cnt - 1,
                            dst0 + CAND + iota)
            pb[v // 8, pl.ds((v % 8) * 16, 16)] = pos
            plsc.addupdate_scatter(offb, [d_aug],
                                   jnp.where(jnp.logical_and(lastm, valid),
                                             cnt, 0))
            return _
        lax.fori_loop(0, nl * 8, _rank, None)

        def _scat2(c, _):
            pltpu.sync_copy(ub.at[pl.ds(c * 128, 128)], cu.at[pb.at[c]])
            pltpu.sync_copy(ib.at[pl.ds(c * 128, 128)], ci.at[pb.at[c]])
            return _
        lax.fori_loop(0, nl, _scat2, None)
        plsc.subcore_barrier()
        return _
    lax.fori_loop(0, 4, _pass, None)

    base = wid * 320
    pltpu.sync_copy(cu.at[pl.ds(base, 320)], ub.at[pl.ds(0, 320)])
    pltpu.sync_copy(ci.at[pl.ds(base, 320)], ib.at[pl.ds(0, 320)])

    def _conv(v, _):
        uk = ub[pl.ds(v * 16, 16)]
        kk = plsc.bitcast(uk ^ jnp.uint32(0x80000000), jnp.int32)
        bits = jnp.where(kk >= 0, kk, kk ^ jnp.int32(0x7FFFFFFF))
        fb[pl.ds(v * 16, 16)] = plsc.bitcast(bits, jnp.float32)
        gpos = base + v * 16 + iota
        iv = ib[pl.ds(v * 16, 16)]
        ib[pl.ds(v * 16, 16)] = jnp.where(gpos < K, iv, 0)
        return _
    lax.fori_loop(0, 20, _conv, None)

    @pl.when(wid < 15)
    def _():
        pltpu.sync_copy(fb, oval_hbm.at[pl.ds(base, 320)])
        pltpu.sync_copy(ib.at[pl.ds(0, 320)], oidx_hbm.at[pl.ds(base, 320)])

    @pl.when(wid == 15)
    def _():
        pltpu.sync_copy(fb.at[pl.ds(0, 200)], oval_hbm.at[pl.ds(4800, 200)])
        pltpu.sync_copy(ib.at[pl.ds(0, 200)], oidx_hbm.at[pl.ds(4800, 200)])


def _k2(keys):
    mesh = plsc.VectorSubcoreMesh(
        core_axis_name="c", subcore_axis_name="s", num_cores=1)
    f = functools.partial(
        pl.kernel,
        mesh=mesh,
        compiler_params=pltpu.CompilerParams(needs_layout_passes=False),
        out_type=[jax.ShapeDtypeStruct((K,), jnp.int32),
                  jax.ShapeDtypeStruct((K,), jnp.float32)],
        scratch_types=[
            pltpu.VMEM((CHUNK,), jnp.int32),
            pltpu.VMEM((SCH,), jnp.uint32),
            pltpu.VMEM((SCH,), jnp.int32),
            pltpu.VMEM((320,), jnp.float32),
            pltpu.VMEM((PBROWS, 128), jnp.int32),
            pltpu.VMEM((32, 128), jnp.int32),
            pltpu.VMEM((BINS,), jnp.int32),
            pltpu.VMEM((16, 256), jnp.int32),
            pltpu.VMEM((16, 16), jnp.int32),
            pltpu.VMEM((272,), jnp.int32),
            pltpu.VMEM((272,), jnp.int32),
            pltpu.VMEM((256,), jnp.int32),
            pltpu.VMEM((256,), jnp.int32),
            pltpu.VMEM((16,), jnp.int32),
            pltpu.VMEM_SHARED((BINS,), jnp.int32),
            pltpu.VMEM_SHARED((16, 256), jnp.int32),
            pltpu.VMEM_SHARED((16, 16), jnp.int32),
            pltpu.VMEM_SHARED((2 * BUF,), jnp.uint32),
            pltpu.VMEM_SHARED((2 * BUF,), jnp.int32),
        ],
    )(_k2_body)
    return f(keys)


def _k3_body(emb_hbm, idx_hbm, out_hbm, ix40, rb, sem):
    wid = lax.axis_index("c") * 16 + lax.axis_index("s")
    iota = _iota16()
    for c in range(4):
        ci = wid * 4 + c

        @pl.when(ci < K // 40)
        def _():
            pltpu.sync_copy(idx_hbm.at[pl.ds(ci * 40, 40)],
                            ix40.at[pl.ds(0, 40)])
            copies = []
            for r in range(40):
                v = ix40[pl.ds((r // 16) * 16, 16)]
                row = _smax(jnp.where(iota == (r % 16), v, 0))
                copies.append(
                    pltpu.async_copy(emb_hbm.at[row], rb.at[r], sem))
            for d in copies:
                d.wait()
            pltpu.sync_copy(rb, out_hbm.at[pl.ds(ci * 40, 40)])


def _k3(emb, idx):
    mesh = plsc.VectorSubcoreMesh(
        core_axis_name="c", subcore_axis_name="s", num_cores=2)
    f = functools.partial(
        pl.kernel,
        mesh=mesh,
        compiler_params=pltpu.CompilerParams(needs_layout_passes=False,
                                             use_tc_tiling_on_sc=True),
        out_type=jax.ShapeDtypeStruct((K, F), jnp.float32),
        scratch_types=[
            pltpu.VMEM((48,), jnp.int32),
            pltpu.VMEM((40, F), jnp.float32),
            pltpu.SemaphoreType.DMA,
        ],
    )(_k3_body)
    return f(emb, idx)


def _k4_body(r_ref, v_ref, o_ref):
    g = jnp.tanh(v_ref[...])
    o_ref[...] = jnp.transpose(r_ref[...] * g, (1, 0))


def _k4(rows, vals):
    return pl.pallas_call(
        _k4_body,
        in_specs=[
            pl.BlockSpec((K, F), lambda: (0, 0)),
            pl.BlockSpec((K, 1), lambda: (0, 0)),
        ],
        out_specs=pl.BlockSpec((F, K), lambda: (0, 0)),
        out_shape=jax.ShapeDtypeStruct((F, K), jnp.float32),
    )(rows, vals)


def kernel(node_embs, scorer):
    norm = jnp.linalg.norm(scorer).reshape(1, 1)
    keys2d, xcopy = _k1(node_embs.T, scorer, norm)
    oidx, oval = _k2(keys2d.reshape(NPAD))
    rows = _k3(xcopy, oidx)
    return _k4(rows, oval.reshape(K, 1))
